# Initial kernel scaffold; baseline (speedup 1.0000x reference)
import jax
import jax.numpy as jnp
from jax.experimental import pallas as pl


def kernel(node_ids, edge_index, fc_weight, attn_l, attn_r):
    raise NotImplementedError("write your pallas kernel here")


# speedup vs baseline: 151.8669x; 151.8669x over previous
import functools

import jax
import jax.numpy as jnp
from jax import lax
from jax.experimental import pallas as pl
from jax.experimental.pallas import tpu as pltpu
from jax.experimental.pallas import tpu_sc as plsc

N = 50000
E = 800000
C = 128
HEADS = 4
HID = 16
NEG_SLOPE = 0.2

NC = 2
NS = 16
CH = 2048
NCHUNKS = 400
E_PAD = NCHUNKS * CH
K_PER_TILE = NCHUNKS // NS
HALF = N // NC
WPN = C // 2
SC_WORDS = HALF * WPN
TILE_WORDS = SC_WORDS // NS
ZB = 4000

TILE_N = 1000
GRID_N = N // TILE_N


def _sc_hist_body(src_hbm, dst_hbm, nid_hbm, cnt_hbm,
                  src_v, dst_v, csb, idxb, valb, zbuf, cntsh, semg, sems):
    cid = lax.axis_index("c")
    sid = lax.axis_index("s")
    lo = cid * HALF

    def _zfill(i, _):
        zbuf[pl.ds(i * 16, 16)] = jnp.zeros((16,), jnp.int32)
        return 0
    lax.fori_loop(0, ZB // 16, _zfill, 0)

    def _zcopy(i, _):
        pltpu.sync_copy(zbuf, cntsh.at[pl.ds(sid * TILE_WORDS + i * ZB, ZB)])
        return 0
    lax.fori_loop(0, TILE_WORDS // ZB, _zcopy, 0)

    plsc.subcore_barrier()

    def _chunk(kk, _):
        row0 = (kk * NS + sid) * NS
        pltpu.sync_copy(src_hbm.at[pl.ds(row0, NS)], src_v)
        pltpu.sync_copy(dst_hbm.at[pl.ds(row0, NS)], dst_v)

        gh = [pltpu.async_copy(nid_hbm.at[src_v.at[j]], csb.at[j], semg)
              for j in range(NS)]
        for h in gh:
            h.wait()

        sh = []
        for j in range(NS):
            def _group(g, _, j=j):
                off = g * 16
                dl = dst_v[j, pl.ds(off, 16)] - lo
                cs = csb[j, pl.ds(off, 16)]
                m = (dl >= 0) & (dl < HALF)
                wi = (dl << 6) + (cs >> 1)
                val = jnp.where(m, jnp.where((cs & 1) == 1, 65536, 1), 0)
                idxb[j, pl.ds(off, 16)] = jnp.where(m, wi, 0)
                valb[j, pl.ds(off, 16)] = val
                return 0
            lax.fori_loop(0, 8, _group, 0)
            sh.append(
                pltpu.async_copy(valb.at[j], cntsh.at[idxb.at[j]], sems,
                                 add=True))
        for h in sh:
            h.wait()
        return 0
    lax.fori_loop(0, K_PER_TILE, _chunk, 0)

    plsc.subcore_barrier()

    def _dump(i, _):
        off = sid * TILE_WORDS + i * ZB
        pltpu.sync_copy(cntsh.at[pl.ds(off, ZB)], zbuf)
        pltpu.sync_copy(zbuf, cnt_hbm.at[pl.ds(cid * SC_WORDS + off, ZB)])
        return 0
    lax.fori_loop(0, TILE_WORDS // ZB, _dump, 0)


def _sc_histogram(src_p, dst_p, node_ids):
    mesh = plsc.VectorSubcoreMesh(core_axis_name="c", subcore_axis_name="s")
    return pl.kernel(
        _sc_hist_body,
        out_type=jax.ShapeDtypeStruct((N * WPN,), jnp.int32),
        mesh=mesh,
        compiler_params=pltpu.CompilerParams(needs_layout_passes=False),
        scratch_types=[
            pltpu.VMEM((NS, 128), jnp.int32),
            pltpu.VMEM((NS, 128), jnp.int32),
            pltpu.VMEM((NS, 128), jnp.int32),
            pltpu.VMEM((NS, 128), jnp.int32),
            pltpu.VMEM((NS, 128), jnp.int32),
            pltpu.VMEM((ZB,), jnp.int32),
            pltpu.VMEM_SHARED((SC_WORDS,), jnp.int32),
            pltpu.SemaphoreType.DMA,
            pltpu.SemaphoreType.DMA,
        ],
    )(src_p, dst_p, node_ids)


def _tc_body(cnt_ref, cls_ref, ex_ref, ft_ref, out_ref):
    cw = cnt_ref[...]
    low = (cw & 0xFFFF).astype(jnp.float32)
    high = (cw >> 16).astype(jnp.float32)
    cnt = jnp.concatenate([low, high], axis=1)
    k = cls_ref[0, 0, :]
    iota = lax.broadcasted_iota(jnp.int32, (TILE_N, C), 1)
    oh = (k[:, None] == iota).astype(jnp.float32)
    ex = jnp.dot(oh, ex_ref[...], preferred_element_type=jnp.float32)
    u = jnp.concatenate([cnt, cnt, cnt, cnt], axis=1) * ex
    y = jnp.dot(u, ft_ref[...], preferred_element_type=jnp.float32)
    num = y[:, :HEADS * HID]
    den = y[:, HEADS * HID:HEADS * HID + HEADS]
    denr = jnp.broadcast_to(den[:, :, None], (TILE_N, HEADS, HID))
    denr = denr.reshape(TILE_N, HEADS * HID)
    good = denr > 0.0
    r = jnp.where(good, num / jnp.where(good, denr, 1.0), 0.0)
    out_ref[...] = jnp.where(r > 0.0, r, jnp.exp(jnp.minimum(r, 0.0)) - 1.0)


def _tc_finish(cnt_w, cls3, exptab2, ft2):
    return pl.pallas_call(
        _tc_body,
        grid=(GRID_N,),
        in_specs=[
            pl.BlockSpec((TILE_N, WPN), lambda i: (i, 0)),
            pl.BlockSpec((1, 1, TILE_N), lambda i: (i, 0, 0)),
            pl.BlockSpec((C, HEADS * C), lambda i: (0, 0)),
            pl.BlockSpec((HEADS * C, 128), lambda i: (0, 0)),
        ],
        out_specs=pl.BlockSpec((TILE_N, HEADS * HID), lambda i: (i, 0)),
        out_shape=jax.ShapeDtypeStruct((N, HEADS * HID), jnp.float32),
    )(cnt_w, cls3, exptab2, ft2)


def kernel(node_ids, edge_index, fc_weight, attn_l, attn_r):
    node_ids = node_ids.astype(jnp.int32)
    ft3 = fc_weight.T.reshape(C, HEADS, HID)
    el_tab = jnp.sum(ft3 * attn_l[None, :, :], axis=-1)
    er_tab = jnp.sum(ft3 * attn_r[None, :, :], axis=-1)
    e_tab = jax.nn.leaky_relu(
        el_tab[:, None, :] + er_tab[None, :, :], negative_slope=NEG_SLOPE)
    etab = jnp.exp(e_tab)
    perm = jnp.concatenate(
        [jnp.arange(0, C, 2), jnp.arange(1, C, 2)]).astype(jnp.int32)
    etp = etab[perm]
    exptab2 = jnp.transpose(etp, (1, 2, 0)).reshape(C, HEADS * C)
    ftp = ft3[perm]
    eye = jnp.eye(HEADS, dtype=jnp.float32)
    blocks = ftp[:, None, :, :] * eye[None, :, :, None]
    ft_part = jnp.transpose(blocks, (1, 0, 2, 3)).reshape(
        HEADS * C, HEADS * HID)
    den_part = jnp.tile(eye, (C, 1)).reshape(C, HEADS, HEADS)
    den_part = jnp.transpose(den_part, (1, 0, 2)).reshape(HEADS * C, HEADS)
    pad = jnp.zeros((HEADS * C, 128 - HEADS * HID - HEADS), jnp.float32)
    ft2 = jnp.concatenate([ft_part, den_part, pad], axis=1)

    src = edge_index[0].astype(jnp.int32)
    dst = edge_index[1].astype(jnp.int32)
    src_p = jnp.concatenate(
        [src, jnp.zeros((E_PAD - E,), jnp.int32)]).reshape(NCHUNKS * NS, 128)
    dst_p = jnp.concatenate(
        [dst, jnp.full((E_PAD - E,), N, jnp.int32)]).reshape(NCHUNKS * NS, 128)

    cnt_w = _sc_histogram(src_p, dst_p, node_ids).reshape(N, WPN)
    cls3 = node_ids.reshape(GRID_N, 1, TILE_N)
    return _tc_finish(cnt_w, cls3, exptab2, ft2)
